# baseline (speedup 1.0000x reference)
import functools

import jax
import jax.numpy as jnp
from jax import lax
from jax.experimental import pallas as pl
from jax.experimental.pallas import tpu as pltpu
from jax.experimental.pallas import tpu_sc as plsc

_SC_ROWS = 512
_CHUNK = 16
_NBUF = 3


def _copy_block(src_ref, out_ref):
    out_ref[...] = src_ref[...]


def _tc_copy(pos_emb, rows, dim):
    block = 512
    return pl.pallas_call(
        _copy_block,
        grid=(rows // block,),
        in_specs=[pl.BlockSpec((block, dim), lambda i: (i, 0))],
        out_specs=pl.BlockSpec((block, dim), lambda i: (i, 0)),
        out_shape=jax.ShapeDtypeStruct((rows, dim), pos_emb.dtype),
    )(pos_emb)


def _sc_copy(pos_emb, start, rows, dim):
    info = plsc.get_sparse_core_info()
    num_workers = info.num_cores * info.num_subcores
    rows_per_w = rows // num_workers
    n_chunks = max(1, rows_per_w // _CHUNK)
    chunk = min(_CHUNK, rows_per_w)

    mesh = plsc.VectorSubcoreMesh(core_axis_name="c", subcore_axis_name="s")

    @functools.partial(
        pl.kernel,
        mesh=mesh,
        out_type=jax.ShapeDtypeStruct((rows, dim), pos_emb.dtype),
        scratch_types=[
            pltpu.VMEM((_NBUF, chunk, dim), pos_emb.dtype),
            pltpu.SemaphoreType.DMA((_NBUF,)),
            pltpu.SemaphoreType.DMA((_NBUF,)),
        ],
    )
    def copy_k(pos_hbm, out_hbm, buf, insem, outsem):
        wid = lax.axis_index("s") * info.num_cores + lax.axis_index("c")
        base = wid * rows_per_w
        ins = []
        outs = []
        for i in range(n_chunks):
            b = i % _NBUF
            ins.append(
                pltpu.make_async_copy(
                    pos_hbm.at[pl.ds(start + base + i * chunk, chunk), :],
                    buf.at[b],
                    insem.at[b],
                )
            )
            outs.append(
                pltpu.make_async_copy(
                    buf.at[b],
                    out_hbm.at[pl.ds(base + i * chunk, chunk), :],
                    outsem.at[b],
                )
            )
        for i in range(min(_NBUF, n_chunks)):
            ins[i].start()
        for i in range(n_chunks):
            ins[i].wait()
            outs[i].start()
            j = i + _NBUF
            if j < n_chunks:
                outs[i].wait()
                ins[j].start()
        for i in range(max(0, n_chunks - _NBUF), n_chunks):
            outs[i].wait()

    return copy_k(pos_emb)


def kernel(x, pos_emb):
    seq_len = x.shape[1]
    dim = pos_emb.shape[1]
    tc_rows = seq_len - _SC_ROWS
    tc_part = _tc_copy(pos_emb, tc_rows, dim)
    sc_part = _sc_copy(pos_emb, tc_rows, _SC_ROWS, dim)
    return jnp.concatenate([tc_part, sc_part], axis=0)


# speedup vs baseline: 2.8495x; 2.8495x over previous
import jax
import jax.numpy as jnp
from jax.experimental import pallas as pl


def _copy_block(src_ref, out_ref):
    out_ref[...] = src_ref[...]


def kernel(x, pos_emb):
    seq_len = x.shape[1]
    dim = pos_emb.shape[1]
    block = 1024
    grid = (seq_len // block,)
    return pl.pallas_call(
        _copy_block,
        grid=grid,
        in_specs=[pl.BlockSpec((block, dim), lambda i: (i, 0))],
        out_specs=pl.BlockSpec((block, dim), lambda i: (i, 0)),
        out_shape=jax.ShapeDtypeStruct((seq_len, dim), pos_emb.dtype),
    )(pos_emb)
